# baseline (speedup 1.0000x reference)
import jax
import jax.numpy as jnp
from jax import lax
from jax.experimental import pallas as pl
from jax.experimental.pallas import tpu as pltpu
from jax.experimental.pallas import tpu_sc as plsc

NN = 10000
NP = 10240
EE = 320000
FF = 16
NC = 2
NS = 16
NW = NC * NS
EPW = EE // NW
CH = 125
NCH = EPW // CH
RPT = NP // NS
NR = NP // 16
SPR = NR // NS
NID = NR // 128

_mesh = plsc.VectorSubcoreMesh(
    core_axis_name="c", subcore_axis_name="s", num_cores=NC, num_subcores=NS
)
_sc_params = pltpu.CompilerParams(
    use_tc_tiling_on_sc=False, needs_layout_passes=False
)
_sc_params_nl = _sc_params

KG = 4
NG = NCH // KG


def _rsqrt_nr(d):
    i = plsc.bitcast(d, jnp.int32)
    y = plsc.bitcast(
        jnp.full((16,), 0x5F3759DF, jnp.int32) - lax.shift_right_logical(i, 1),
        jnp.float32,
    )
    for _ in range(3):
        y = y * (1.5 - 0.5 * d * y * y)
    return y


def _dinv_packed(dp0_v, dp1_v, dpk_v):
    @pl.loop(0, SPR)
    def _(r):
        dpk_v[r, :] = _rsqrt_nr(1.0 + dp0_v[r] + dp1_v[r])


def _edge_pipeline(gsrc, src_v, dst_v, rows_v, acc_sh, gsem0, gsem1):
    def fire(g, half, sem):
        for b in range(KG):
            pltpu.async_copy(
                gsrc.at[src_v.at[g * KG + b]], rows_v.at[half * KG + b], sem
            )

    def drain(g, half, sem):
        for b in range(KG):
            pltpu.make_async_copy(
                gsrc.at[src_v.at[g * KG + b]], rows_v.at[half * KG + b], sem
            ).wait()

    def scat(g, half):
        for b in range(KG):
            pltpu.sync_copy(
                rows_v.at[half * KG + b], acc_sh.at[dst_v.at[g * KG + b]],
                add=True,
            )

    fire(0, 0, gsem0)

    @pl.loop(0, NG, step=2)
    def _pair(g):
        fire(g + 1, 1, gsem1)
        drain(g, 0, gsem0)
        scat(g, 0)

        @pl.when(g + 2 < NG)
        def _():
            fire(g + 2, 0, gsem0)

        drain(g + 1, 1, gsem1)
        scat(g + 1, 1)


def _mm1_body(x_ref, w1_ref, h1_ref):
    h1_ref[pl.ds(0, NN), :] = jnp.dot(
        x_ref[...], w1_ref[...], preferred_element_type=jnp.float32
    )
    h1_ref[pl.ds(NN, NP - NN), :] = jnp.zeros((NP - NN, FF), jnp.float32)


_mm1 = pl.pallas_call(
    _mm1_body, out_shape=jax.ShapeDtypeStruct((NP, FF), jnp.float32)
)


def _deg_body(edge_hbm, iden_hbm, zeros_hbm, out_hbm, dst_v, iden_v, loc_v,
              acc_sh, ssem):
    cid = lax.axis_index("c")
    sid = lax.axis_index("s")
    wid = sid * NC + cid
    pltpu.sync_copy(edge_hbm.at[1, pl.ds(wid * EPW, EPW)], dst_v)
    pltpu.sync_copy(iden_hbm, iden_v)
    pltpu.sync_copy(zeros_hbm, acc_sh.at[pl.ds(sid * SPR, SPR)])

    @pl.loop(0, NR)
    def _zero(i):
        loc_v[i, :] = jnp.zeros((16,), jnp.float32)

    ones16 = jnp.ones((16,), jnp.float32)

    @pl.loop(0, EPW // 16)
    def _count(i):
        dv = dst_v[pl.ds(i * 16, 16)]
        plsc.addupdate_scatter(
            loc_v,
            [lax.shift_right_logical(dv, 4), lax.bitwise_and(dv, 15)],
            ones16,
        )

    plsc.subcore_barrier()

    for j in range(NID):
        pltpu.async_copy(
            loc_v.at[pl.ds(j * 128, 128)], acc_sh.at[iden_v.at[j]], ssem,
            add=True,
        )
    for j in range(NID):
        pltpu.make_async_copy(
            loc_v.at[pl.ds(j * 128, 128)], acc_sh.at[iden_v.at[j]], ssem
        ).wait()

    plsc.subcore_barrier()
    pltpu.sync_copy(
        acc_sh.at[pl.ds(sid * SPR, SPR)],
        out_hbm.at[cid, pl.ds(sid * SPR, SPR)],
    )


_deg_call = pl.kernel(
    _deg_body,
    out_type=jax.ShapeDtypeStruct((NC, NR, 16), jnp.float32),
    mesh=_mesh,
    scratch_types=[
        pltpu.VMEM((EPW,), jnp.int32),
        pltpu.VMEM((NID, 128), jnp.int32),
        pltpu.VMEM((NR, 16), jnp.float32),
        pltpu.VMEM_SHARED((NR, 16), jnp.float32),
        pltpu.SemaphoreType.DMA,
    ],
    compiler_params=_sc_params_nl,
)


def _agg1_body(edge_hbm, h1_hbm, degp_hbm, zeros_hbm, aggp_hbm, g1_hbm,
               src_v, dst_v, rows_v, dp0_v, dp1_v, dpk_v, g1_v, acc_sh, g_sh,
               gsem0, gsem1):
    cid = lax.axis_index("c")
    sid = lax.axis_index("s")
    wid = sid * NC + cid
    pltpu.sync_copy(edge_hbm.at[0, wid], src_v)
    pltpu.sync_copy(edge_hbm.at[1, wid], dst_v)
    pltpu.sync_copy(zeros_hbm, acc_sh.at[pl.ds(sid * RPT, RPT)])
    pltpu.sync_copy(degp_hbm.at[0, pl.ds(sid * SPR, SPR)], dp0_v)
    pltpu.sync_copy(degp_hbm.at[1, pl.ds(sid * SPR, SPR)], dp1_v)
    pltpu.sync_copy(h1_hbm.at[pl.ds(sid * RPT, RPT)], g1_v)
    _dinv_packed(dp0_v, dp1_v, dpk_v)

    @pl.loop(0, SPR)
    def _scale(r):
        drow = dpk_v[r, :]
        for c in range(16):
            n = r * 16 + c
            g1_v[n, :] = g1_v[n, :] * drow[c]

    pltpu.sync_copy(g1_v, g_sh.at[pl.ds(sid * RPT, RPT)])
    pltpu.sync_copy(g1_v, g1_hbm.at[pl.ds(sid * RPT, RPT)])
    plsc.subcore_barrier()

    _edge_pipeline(g_sh, src_v, dst_v, rows_v, acc_sh, gsem0, gsem1)

    plsc.subcore_barrier()
    pltpu.sync_copy(
        acc_sh.at[pl.ds(sid * RPT, RPT)],
        aggp_hbm.at[cid, pl.ds(sid * RPT, RPT)],
    )


_agg1_call = pl.kernel(
    _agg1_body,
    out_type=[
        jax.ShapeDtypeStruct((NC, NP, FF), jnp.float32),
        jax.ShapeDtypeStruct((NP, FF), jnp.float32),
    ],
    mesh=_mesh,
    scratch_types=[
        pltpu.VMEM((NCH, CH), jnp.int32),
        pltpu.VMEM((NCH, CH), jnp.int32),
        pltpu.VMEM((2 * KG, CH, FF), jnp.float32),
        pltpu.VMEM((SPR, 16), jnp.float32),
        pltpu.VMEM((SPR, 16), jnp.float32),
        pltpu.VMEM((SPR, 16), jnp.float32),
        pltpu.VMEM((RPT, FF), jnp.float32),
        pltpu.VMEM_SHARED((NP, FF), jnp.float32),
        pltpu.VMEM_SHARED((NP, FF), jnp.float32),
        pltpu.SemaphoreType.DMA,
        pltpu.SemaphoreType.DMA,
    ],
    compiler_params=_sc_params,
)


def _agg2_body(edge_hbm, degp_hbm, aggp1_hbm, g1_hbm, b1_hbm, w2_hbm,
               zeros_hbm, aggp_hbm, g2_hbm,
               src_v, dst_v, rows_v, dp0_v, dp1_v, dpk_v, p0_v, p1_v, g_v,
               b1_v, w2_v, acc_sh, g_sh, gsem0, gsem1):
    cid = lax.axis_index("c")
    sid = lax.axis_index("s")
    wid = sid * NC + cid
    pltpu.sync_copy(edge_hbm.at[0, wid], src_v)
    pltpu.sync_copy(edge_hbm.at[1, wid], dst_v)
    pltpu.sync_copy(zeros_hbm, acc_sh.at[pl.ds(sid * RPT, RPT)])
    pltpu.sync_copy(degp_hbm.at[0, pl.ds(sid * SPR, SPR)], dp0_v)
    pltpu.sync_copy(degp_hbm.at[1, pl.ds(sid * SPR, SPR)], dp1_v)
    pltpu.sync_copy(aggp1_hbm.at[0, pl.ds(sid * RPT, RPT)], p0_v)
    pltpu.sync_copy(aggp1_hbm.at[1, pl.ds(sid * RPT, RPT)], p1_v)
    pltpu.sync_copy(g1_hbm.at[pl.ds(sid * RPT, RPT)], g_v)
    pltpu.sync_copy(b1_hbm, b1_v)
    pltpu.sync_copy(w2_hbm, w2_v)
    _dinv_packed(dp0_v, dp1_v, dpk_v)

    b1vec = b1_v[...]
    w2rows = [w2_v[k, :] for k in range(FF)]

    @pl.loop(0, SPR)
    def _comb(r):
        drow = dpk_v[r, :]
        for c in range(16):
            n = r * 16 + c
            row = jnp.maximum(
                b1vec + drow[c] * (p0_v[n, :] + p1_v[n, :] + g_v[n, :]), 0.0
            )
            h2 = row[0] * w2rows[0]
            for k in range(1, FF):
                h2 = h2 + row[k] * w2rows[k]
            g_v[n, :] = h2 * drow[c]

    pltpu.sync_copy(g_v, g_sh.at[pl.ds(sid * RPT, RPT)])
    pltpu.sync_copy(g_v, g2_hbm.at[pl.ds(sid * RPT, RPT)])
    plsc.subcore_barrier()

    _edge_pipeline(g_sh, src_v, dst_v, rows_v, acc_sh, gsem0, gsem1)

    plsc.subcore_barrier()
    pltpu.sync_copy(
        acc_sh.at[pl.ds(sid * RPT, RPT)],
        aggp_hbm.at[cid, pl.ds(sid * RPT, RPT)],
    )


_agg2_call = pl.kernel(
    _agg2_body,
    out_type=[
        jax.ShapeDtypeStruct((NC, NP, FF), jnp.float32),
        jax.ShapeDtypeStruct((NP, FF), jnp.float32),
    ],
    mesh=_mesh,
    scratch_types=[
        pltpu.VMEM((NCH, CH), jnp.int32),
        pltpu.VMEM((NCH, CH), jnp.int32),
        pltpu.VMEM((2 * KG, CH, FF), jnp.float32),
        pltpu.VMEM((SPR, 16), jnp.float32),
        pltpu.VMEM((SPR, 16), jnp.float32),
        pltpu.VMEM((SPR, 16), jnp.float32),
        pltpu.VMEM((RPT, FF), jnp.float32),
        pltpu.VMEM((RPT, FF), jnp.float32),
        pltpu.VMEM((RPT, FF), jnp.float32),
        pltpu.VMEM((16,), jnp.float32),
        pltpu.VMEM((FF, FF), jnp.float32),
        pltpu.VMEM_SHARED((NP, FF), jnp.float32),
        pltpu.VMEM_SHARED((NP, FF), jnp.float32),
        pltpu.SemaphoreType.DMA,
        pltpu.SemaphoreType.DMA,
    ],
    compiler_params=_sc_params,
)


def _fin_body(degp_hbm, aggp2_hbm, g2_hbm, b2_hbm, fcv_hbm, out_hbm,
              dp0_v, dp1_v, dpk_v, p0_v, p1_v, g_v, b2_v, fcv_v,
              tot_v, pool_sh):
    cid = lax.axis_index("c")
    sid = lax.axis_index("s")
    pltpu.sync_copy(degp_hbm.at[0, pl.ds(sid * SPR, SPR)], dp0_v)
    pltpu.sync_copy(degp_hbm.at[1, pl.ds(sid * SPR, SPR)], dp1_v)
    pltpu.sync_copy(aggp2_hbm.at[0, pl.ds(sid * RPT, RPT)], p0_v)
    pltpu.sync_copy(aggp2_hbm.at[1, pl.ds(sid * RPT, RPT)], p1_v)
    pltpu.sync_copy(g2_hbm.at[pl.ds(sid * RPT, RPT)], g_v)
    pltpu.sync_copy(b2_hbm, b2_v)
    pltpu.sync_copy(fcv_hbm, fcv_v)
    _dinv_packed(dp0_v, dp1_v, dpk_v)

    lanes = lax.iota(jnp.int32, 16)
    mask8 = lanes < 8
    b2vec = jnp.where(mask8, plsc.load_gather(b2_v, [jnp.minimum(lanes, 7)]),
                      0.0)
    nrows = jnp.where(sid == NS - 1, (NN - (NS - 1) * RPT) // 16, SPR)

    def _body(r, acc):
        drow = dpk_v[r, :]
        for c in range(16):
            n = r * 16 + c
            row = jnp.maximum(
                b2vec + drow[c] * (p0_v[n, :] + p1_v[n, :] + g_v[n, :]), 0.0
            )
            acc = acc + row
        return acc

    pooled = lax.fori_loop(0, nrows, _body, jnp.zeros((16,), jnp.float32))
    tot_v[...] = pooled
    pltpu.sync_copy(tot_v, pool_sh.at[sid])
    plsc.subcore_barrier()

    @pl.when(sid == 0)
    def _():
        pltpu.sync_copy(pool_sh, g_v.at[pl.ds(0, NS)])
        total = jnp.zeros((16,), jnp.float32)
        for r in range(NS):
            total = total + g_v[r, :]
        total = total * (1.0 / NN)
        fcv = fcv_v[...]
        z = lax.reduce_sum_p.bind(
            total * jnp.where(mask8, fcv, 0.0), axes=(0,)
        ) + fcv[8]
        zv = jnp.zeros((16,), jnp.float32) + z
        sig = 1.0 / (1.0 + jnp.exp(-zv))
        tot_v[...] = sig
        pltpu.sync_copy(tot_v, out_hbm.at[cid])


_fin_call = pl.kernel(
    _fin_body,
    out_type=jax.ShapeDtypeStruct((NC, 16), jnp.float32),
    mesh=_mesh,
    scratch_types=[
        pltpu.VMEM((SPR, 16), jnp.float32),
        pltpu.VMEM((SPR, 16), jnp.float32),
        pltpu.VMEM((SPR, 16), jnp.float32),
        pltpu.VMEM((RPT, FF), jnp.float32),
        pltpu.VMEM((RPT, FF), jnp.float32),
        pltpu.VMEM((RPT, FF), jnp.float32),
        pltpu.VMEM((8,), jnp.float32),
        pltpu.VMEM((16,), jnp.float32),
        pltpu.VMEM((16,), jnp.float32),
        pltpu.VMEM_SHARED((NS, 16), jnp.float32),
    ],
    compiler_params=_sc_params,
)


@jax.jit
def kernel(x, edge_index, W1, b1, W2, b2, fcW, fcb):
    edge4 = edge_index.reshape(2, NW, NCH, CH)
    zeros_stripe = jnp.zeros((RPT, FF), jnp.float32)
    zeros_deg = jnp.zeros((SPR, 16), jnp.float32)
    iden = jnp.arange(NR, dtype=jnp.int32).reshape(NID, 128)

    degp = _deg_call(edge_index, iden, zeros_deg)
    h1 = _mm1(x, W1)
    agg1p, g1 = _agg1_call(edge4, h1, degp, zeros_stripe)
    w2p = jnp.pad(W2, ((0, 0), (0, FF - W2.shape[1])))
    agg2p, g2 = _agg2_call(edge4, degp, agg1p, g1, b1, w2p, zeros_stripe)
    fcv = jnp.concatenate([fcW[:, 0], fcb, jnp.zeros((7,), jnp.float32)])
    sig = _fin_call(degp, agg2p, g2, b2, fcv)
    return sig[0, :1]


# speedup vs baseline: 1.0977x; 1.0977x over previous
import jax
import jax.numpy as jnp
from jax import lax
from jax.experimental import pallas as pl
from jax.experimental.pallas import tpu as pltpu
from jax.experimental.pallas import tpu_sc as plsc

NN = 10000
NP = 10240
EE = 320000
FF = 16
NC = 2
NS = 16
NW = NC * NS
EPW = EE // NW
CH = 125
NCH = EPW // CH
RPT = NP // NS
NR = NP // 16
SPR = NR // NS
NID = NR // 128

_mesh = plsc.VectorSubcoreMesh(
    core_axis_name="c", subcore_axis_name="s", num_cores=NC, num_subcores=NS
)
_sc_params = pltpu.CompilerParams(
    use_tc_tiling_on_sc=False, needs_layout_passes=False
)
_sc_params_nl = _sc_params

KG = 4
NG = NCH // KG


def _rsqrt_nr(d):
    i = plsc.bitcast(d, jnp.int32)
    y = plsc.bitcast(
        jnp.full((16,), 0x5F3759DF, jnp.int32) - lax.shift_right_logical(i, 1),
        jnp.float32,
    )
    for _ in range(3):
        y = y * (1.5 - 0.5 * d * y * y)
    return y


def _dinv_packed(dp0_v, dp1_v, dpk_v):
    @pl.loop(0, SPR)
    def _(r):
        dpk_v[r, :] = _rsqrt_nr(1.0 + dp0_v[r] + dp1_v[r])


def _edge_pipeline(gsrc, src_v, dst_v, rows_v, acc_sh, gsems, ssems):
    def fire_g(g, q):
        for b in range(KG):
            pltpu.async_copy(
                gsrc.at[src_v.at[g * KG + b]], rows_v.at[q * KG + b],
                gsems[q],
            )

    def drain_g(g, q):
        for b in range(KG):
            pltpu.make_async_copy(
                gsrc.at[src_v.at[g * KG + b]], rows_v.at[q * KG + b],
                gsems[q],
            ).wait()

    def fire_s(g, q):
        for b in range(KG):
            pltpu.async_copy(
                rows_v.at[q * KG + b], acc_sh.at[dst_v.at[g * KG + b]],
                ssems[q], add=True,
            )

    def drain_s(g, q):
        for b in range(KG):
            pltpu.make_async_copy(
                rows_v.at[q * KG + b], acc_sh.at[dst_v.at[g * KG + b]],
                ssems[q],
            ).wait()

    @pl.loop(0, NG, step=4)
    def _quad(gb):
        for qi in range(4):
            g = gb + qi

            @pl.when(g >= 4)
            def _(g=g, qi=qi):
                drain_s(g - 4, qi)

            fire_g(g, qi)

            @pl.when(g >= 1)
            def _(g=g, qi=qi):
                drain_g(g - 1, (qi - 1) % 4)
                fire_s(g - 1, (qi - 1) % 4)

    drain_g(NG - 1, (NG - 1) % 4)
    fire_s(NG - 1, (NG - 1) % 4)
    for g in range(NG - 4, NG):
        drain_s(g, g % 4)


def _mm1_body(x_ref, w1_ref, h1_ref):
    h1_ref[pl.ds(0, NN), :] = jnp.dot(
        x_ref[...], w1_ref[...], preferred_element_type=jnp.float32
    )
    h1_ref[pl.ds(NN, NP - NN), :] = jnp.zeros((NP - NN, FF), jnp.float32)


_mm1 = pl.pallas_call(
    _mm1_body, out_shape=jax.ShapeDtypeStruct((NP, FF), jnp.float32)
)


def _deg_body(edge_hbm, iden_hbm, zeros_hbm, out_hbm, dst_v, iden_v, loc_v,
              acc_sh, ssem):
    cid = lax.axis_index("c")
    sid = lax.axis_index("s")
    wid = sid * NC + cid
    pltpu.sync_copy(edge_hbm.at[1, wid], dst_v)
    pltpu.sync_copy(iden_hbm, iden_v)
    pltpu.sync_copy(zeros_hbm, acc_sh.at[pl.ds(sid * SPR, SPR)])

    @pl.loop(0, NR)
    def _zero(i):
        loc_v[i, :] = jnp.zeros((16,), jnp.float32)

    ones16 = jnp.ones((16,), jnp.float32)
    lanes = lax.iota(jnp.int32, 16)
    tail_mask = lanes < (CH - 112)

    @pl.loop(0, NCH)
    def _count(j):
        for c in range(0, 112, 16):
            dv = dst_v[j, pl.ds(c, 16)]
            plsc.addupdate_scatter(
                loc_v,
                [lax.shift_right_logical(dv, 4), lax.bitwise_and(dv, 15)],
                ones16,
            )
        dvt = plsc.load_gather(dst_v, [jnp.full((16,), j, jnp.int32),
                                       jnp.minimum(112 + lanes, CH - 1)])
        plsc.addupdate_scatter(
            loc_v,
            [lax.shift_right_logical(dvt, 4), lax.bitwise_and(dvt, 15)],
            ones16, mask=tail_mask,
        )

    plsc.subcore_barrier()

    for j in range(NID):
        pltpu.async_copy(
            loc_v.at[pl.ds(j * 128, 128)], acc_sh.at[iden_v.at[j]], ssem,
            add=True,
        )
    for j in range(NID):
        pltpu.make_async_copy(
            loc_v.at[pl.ds(j * 128, 128)], acc_sh.at[iden_v.at[j]], ssem
        ).wait()

    plsc.subcore_barrier()
    pltpu.sync_copy(
        acc_sh.at[pl.ds(sid * SPR, SPR)],
        out_hbm.at[cid, pl.ds(sid * SPR, SPR)],
    )


_deg_call = pl.kernel(
    _deg_body,
    out_type=jax.ShapeDtypeStruct((NC, NR, 16), jnp.float32),
    mesh=_mesh,
    scratch_types=[
        pltpu.VMEM((NCH, CH), jnp.int32),
        pltpu.VMEM((NID, 128), jnp.int32),
        pltpu.VMEM((NR, 16), jnp.float32),
        pltpu.VMEM_SHARED((NR, 16), jnp.float32),
        pltpu.SemaphoreType.DMA,
    ],
    compiler_params=_sc_params_nl,
)


def _agg1_body(edge_hbm, h1_hbm, degp_hbm, zeros_hbm, aggp_hbm, g1_hbm,
               src_v, dst_v, rows_v, dp0_v, dp1_v, dpk_v, g1_v, acc_sh, g_sh,
               *sems):
    cid = lax.axis_index("c")
    sid = lax.axis_index("s")
    wid = sid * NC + cid
    pltpu.sync_copy(edge_hbm.at[0, wid], src_v)
    pltpu.sync_copy(edge_hbm.at[1, wid], dst_v)
    pltpu.sync_copy(zeros_hbm, acc_sh.at[pl.ds(sid * RPT, RPT)])
    pltpu.sync_copy(degp_hbm.at[0, pl.ds(sid * SPR, SPR)], dp0_v)
    pltpu.sync_copy(degp_hbm.at[1, pl.ds(sid * SPR, SPR)], dp1_v)
    pltpu.sync_copy(h1_hbm.at[pl.ds(sid * RPT, RPT)], g1_v)
    _dinv_packed(dp0_v, dp1_v, dpk_v)

    @pl.loop(0, SPR)
    def _scale(r):
        drow = dpk_v[r, :]
        for c in range(16):
            n = r * 16 + c
            g1_v[n, :] = g1_v[n, :] * drow[c]

    pltpu.sync_copy(g1_v, g_sh.at[pl.ds(sid * RPT, RPT)])
    pltpu.sync_copy(g1_v, g1_hbm.at[pl.ds(sid * RPT, RPT)])
    plsc.subcore_barrier()

    _edge_pipeline(g_sh, src_v, dst_v, rows_v, acc_sh, sems[:4], sems[4:])

    plsc.subcore_barrier()
    pltpu.sync_copy(
        acc_sh.at[pl.ds(sid * RPT, RPT)],
        aggp_hbm.at[cid, pl.ds(sid * RPT, RPT)],
    )


_agg1_call = pl.kernel(
    _agg1_body,
    out_type=[
        jax.ShapeDtypeStruct((NC, NP, FF), jnp.float32),
        jax.ShapeDtypeStruct((NP, FF), jnp.float32),
    ],
    mesh=_mesh,
    scratch_types=[
        pltpu.VMEM((NCH, CH), jnp.int32),
        pltpu.VMEM((NCH, CH), jnp.int32),
        pltpu.VMEM((4 * KG, CH, FF), jnp.float32),
        pltpu.VMEM((SPR, 16), jnp.float32),
        pltpu.VMEM((SPR, 16), jnp.float32),
        pltpu.VMEM((SPR, 16), jnp.float32),
        pltpu.VMEM((RPT, FF), jnp.float32),
        pltpu.VMEM_SHARED((NP, FF), jnp.float32),
        pltpu.VMEM_SHARED((NP, FF), jnp.float32),
    ] + [pltpu.SemaphoreType.DMA] * 8,
    compiler_params=_sc_params,
)


def _agg2_body(edge_hbm, degp_hbm, aggp1_hbm, g1_hbm, b1_hbm, w2_hbm,
               zeros_hbm, aggp_hbm, g2_hbm,
               src_v, dst_v, rows_v, dp0_v, dp1_v, dpk_v, p0_v, p1_v, g_v,
               b1_v, w2_v, acc_sh, g_sh, *sems):
    cid = lax.axis_index("c")
    sid = lax.axis_index("s")
    wid = sid * NC + cid
    pltpu.sync_copy(edge_hbm.at[0, wid], src_v)
    pltpu.sync_copy(edge_hbm.at[1, wid], dst_v)
    pltpu.sync_copy(zeros_hbm, acc_sh.at[pl.ds(sid * RPT, RPT)])
    pltpu.sync_copy(degp_hbm.at[0, pl.ds(sid * SPR, SPR)], dp0_v)
    pltpu.sync_copy(degp_hbm.at[1, pl.ds(sid * SPR, SPR)], dp1_v)
    pltpu.sync_copy(aggp1_hbm.at[0, pl.ds(sid * RPT, RPT)], p0_v)
    pltpu.sync_copy(aggp1_hbm.at[1, pl.ds(sid * RPT, RPT)], p1_v)
    pltpu.sync_copy(g1_hbm.at[pl.ds(sid * RPT, RPT)], g_v)
    pltpu.sync_copy(b1_hbm, b1_v)
    pltpu.sync_copy(w2_hbm, w2_v)
    _dinv_packed(dp0_v, dp1_v, dpk_v)

    b1vec = b1_v[...]
    w2rows = [w2_v[k, :] for k in range(FF)]

    @pl.loop(0, SPR)
    def _comb(r):
        drow = dpk_v[r, :]
        for c in range(16):
            n = r * 16 + c
            row = jnp.maximum(
                b1vec + drow[c] * (p0_v[n, :] + p1_v[n, :] + g_v[n, :]), 0.0
            )
            h2 = row[0] * w2rows[0]
            for k in range(1, FF):
                h2 = h2 + row[k] * w2rows[k]
            g_v[n, :] = h2 * drow[c]

    pltpu.sync_copy(g_v, g_sh.at[pl.ds(sid * RPT, RPT)])
    pltpu.sync_copy(g_v, g2_hbm.at[pl.ds(sid * RPT, RPT)])
    plsc.subcore_barrier()

    _edge_pipeline(g_sh, src_v, dst_v, rows_v, acc_sh, sems[:4], sems[4:])

    plsc.subcore_barrier()
    pltpu.sync_copy(
        acc_sh.at[pl.ds(sid * RPT, RPT)],
        aggp_hbm.at[cid, pl.ds(sid * RPT, RPT)],
    )


_agg2_call = pl.kernel(
    _agg2_body,
    out_type=[
        jax.ShapeDtypeStruct((NC, NP, FF), jnp.float32),
        jax.ShapeDtypeStruct((NP, FF), jnp.float32),
    ],
    mesh=_mesh,
    scratch_types=[
        pltpu.VMEM((NCH, CH), jnp.int32),
        pltpu.VMEM((NCH, CH), jnp.int32),
        pltpu.VMEM((4 * KG, CH, FF), jnp.float32),
        pltpu.VMEM((SPR, 16), jnp.float32),
        pltpu.VMEM((SPR, 16), jnp.float32),
        pltpu.VMEM((SPR, 16), jnp.float32),
        pltpu.VMEM((RPT, FF), jnp.float32),
        pltpu.VMEM((RPT, FF), jnp.float32),
        pltpu.VMEM((RPT, FF), jnp.float32),
        pltpu.VMEM((16,), jnp.float32),
        pltpu.VMEM((FF, FF), jnp.float32),
        pltpu.VMEM_SHARED((NP, FF), jnp.float32),
        pltpu.VMEM_SHARED((NP, FF), jnp.float32),
    ] + [pltpu.SemaphoreType.DMA] * 8,
    compiler_params=_sc_params,
)


def _fin_body(degp_hbm, aggp2_hbm, g2_hbm, b2_hbm, fcv_hbm, out_hbm,
              dp0_v, dp1_v, dpk_v, p0_v, p1_v, g_v, b2_v, fcv_v,
              tot_v, pool_sh):
    cid = lax.axis_index("c")
    sid = lax.axis_index("s")
    pltpu.sync_copy(degp_hbm.at[0, pl.ds(sid * SPR, SPR)], dp0_v)
    pltpu.sync_copy(degp_hbm.at[1, pl.ds(sid * SPR, SPR)], dp1_v)
    pltpu.sync_copy(aggp2_hbm.at[0, pl.ds(sid * RPT, RPT)], p0_v)
    pltpu.sync_copy(aggp2_hbm.at[1, pl.ds(sid * RPT, RPT)], p1_v)
    pltpu.sync_copy(g2_hbm.at[pl.ds(sid * RPT, RPT)], g_v)
    pltpu.sync_copy(b2_hbm, b2_v)
    pltpu.sync_copy(fcv_hbm, fcv_v)
    _dinv_packed(dp0_v, dp1_v, dpk_v)

    lanes = lax.iota(jnp.int32, 16)
    mask8 = lanes < 8
    b2vec = jnp.where(mask8, plsc.load_gather(b2_v, [jnp.minimum(lanes, 7)]),
                      0.0)
    nrows = jnp.where(sid == NS - 1, (NN - (NS - 1) * RPT) // 16, SPR)

    def _body(r, acc):
        drow = dpk_v[r, :]
        for c in range(16):
            n = r * 16 + c
            row = jnp.maximum(
                b2vec + drow[c] * (p0_v[n, :] + p1_v[n, :] + g_v[n, :]), 0.0
            )
            acc = acc + row
        return acc

    pooled = lax.fori_loop(0, nrows, _body, jnp.zeros((16,), jnp.float32))
    tot_v[...] = pooled
    pltpu.sync_copy(tot_v, pool_sh.at[sid])
    plsc.subcore_barrier()

    @pl.when(sid == 0)
    def _():
        pltpu.sync_copy(pool_sh, g_v.at[pl.ds(0, NS)])
        total = jnp.zeros((16,), jnp.float32)
        for r in range(NS):
            total = total + g_v[r, :]
        total = total * (1.0 / NN)
        fcv = fcv_v[...]
        z = lax.reduce_sum_p.bind(
            total * jnp.where(mask8, fcv, 0.0), axes=(0,)
        ) + fcv[8]
        zv = jnp.zeros((16,), jnp.float32) + z
        sig = 1.0 / (1.0 + jnp.exp(-zv))
        tot_v[...] = sig
        pltpu.sync_copy(tot_v, out_hbm.at[cid])


_fin_call = pl.kernel(
    _fin_body,
    out_type=jax.ShapeDtypeStruct((NC, 16), jnp.float32),
    mesh=_mesh,
    scratch_types=[
        pltpu.VMEM((SPR, 16), jnp.float32),
        pltpu.VMEM((SPR, 16), jnp.float32),
        pltpu.VMEM((SPR, 16), jnp.float32),
        pltpu.VMEM((RPT, FF), jnp.float32),
        pltpu.VMEM((RPT, FF), jnp.float32),
        pltpu.VMEM((RPT, FF), jnp.float32),
        pltpu.VMEM((8,), jnp.float32),
        pltpu.VMEM((16,), jnp.float32),
        pltpu.VMEM((16,), jnp.float32),
        pltpu.VMEM_SHARED((NS, 16), jnp.float32),
    ],
    compiler_params=_sc_params,
)


@jax.jit
def kernel(x, edge_index, W1, b1, W2, b2, fcW, fcb):
    edge4 = edge_index.reshape(2, NW, NCH, CH)
    zeros_stripe = jnp.zeros((RPT, FF), jnp.float32)
    zeros_deg = jnp.zeros((SPR, 16), jnp.float32)
    iden = jnp.arange(NR, dtype=jnp.int32).reshape(NID, 128)

    degp = _deg_call(edge4, iden, zeros_deg)
    h1 = _mm1(x, W1)
    agg1p, g1 = _agg1_call(edge4, h1, degp, zeros_stripe)
    w2p = jnp.pad(W2, ((0, 0), (0, FF - W2.shape[1])))
    agg2p, g2 = _agg2_call(edge4, degp, agg1p, g1, b1, w2p, zeros_stripe)
    fcv = jnp.concatenate([fcW[:, 0], fcb, jnp.zeros((7,), jnp.float32)])
    sig = _fin_call(degp, agg2p, g2, b2, fcv)
    return sig[0, :1]
